# baseline (speedup 1.0000x reference)
import functools

import jax
import jax.numpy as jnp
from jax import lax
from jax.experimental import pallas as pl
from jax.experimental.pallas import tpu as pltpu
from jax.experimental.pallas import tpu_sc as plsc

NSTREAM = 8
_NEG = float("-inf")
_POS = float("inf")


def _conv_kernel(*refs, n):
    x_refs = refs[:NSTREAM]
    w_refs = refs[NSTREAM:2 * NSTREAM]
    cb_ref, out_ref = refs[2 * NSTREAM:]

    t = jnp.zeros((3, n), jnp.float32)
    for s in range(NSTREAM):
        t = t + jnp.dot(w_refs[s][...], x_refs[s][0, 0],
                        preferred_element_type=jnp.float32)

    t0 = t[0:1, :]
    t1 = t[1:2, :]
    t2 = t[2:3, :]
    zero = jnp.zeros((1, 1), jnp.float32)
    y = t1 + cb_ref[0, 0]
    y = y + jnp.concatenate([zero, t0[:, : n - 1]], axis=1)
    y = y + jnp.concatenate([t2[:, 1:], zero], axis=1)
    out_ref[...] = y.reshape(1, 1, n)


def _conv(x, conv_w, conv_b):
    B, C, N = x.shape
    cs = C // NSTREAM
    xs = x.reshape(B, NSTREAM, cs, N)
    wt = conv_w[0].T

    def x_spec(s):
        return pl.BlockSpec((1, 1, cs, N), lambda b, s=s: (b, s, 0, 0))

    def w_spec(s):
        return pl.BlockSpec((3, cs), lambda b, s=s: (0, s))

    in_specs = ([x_spec(s) for s in range(NSTREAM)]
                + [w_spec(s) for s in range(NSTREAM)]
                + [pl.BlockSpec((1, 1), lambda b: (0, 0))])
    operands = [xs] * NSTREAM + [wt] * NSTREAM + [conv_b.reshape(1, 1)]
    y = pl.pallas_call(
        functools.partial(_conv_kernel, n=N),
        grid=(B,),
        in_specs=in_specs,
        out_specs=pl.BlockSpec((1, 1, N), lambda b: (b, 0, 0)),
        out_shape=jax.ShapeDtypeStruct((B, 1, N), jnp.float32),
        compiler_params=pltpu.CompilerParams(
            dimension_semantics=("arbitrary",)),
    )(*operands)
    return y.reshape(B, N)


def _topk_sc_body(y_hbm, out_hbm, yv, fv, n=2048, nb=8):
    wid = lax.axis_index("s") * 2 + lax.axis_index("c")

    @pl.when(wid < nb)
    def _():
        pltpu.sync_copy(y_hbm.at[wid], yv)

        def step(i, carry):
            mx1, mx2, mn1, mn2 = carry
            v = yv[pl.ds(i * 16, 16)]
            nmx1 = jnp.maximum(mx1, v)
            nmx2 = jnp.maximum(mx2, jnp.minimum(mx1, v))
            nmn1 = jnp.minimum(mn1, v)
            nmn2 = jnp.minimum(mn2, jnp.maximum(mn1, v))
            return nmx1, nmx2, nmn1, nmn2

        neg = jnp.full((16,), _NEG, jnp.float32)
        pos = jnp.full((16,), _POS, jnp.float32)
        mx1, mx2, mn1, mn2 = lax.fori_loop(0, n // 16, step,
                                           (neg, neg, pos, pos))
        fv[pl.ds(0, 16)] = mx1
        fv[pl.ds(16, 16)] = mx2
        fv[pl.ds(32, 16)] = mn1
        fv[pl.ds(48, 16)] = mn2
        pltpu.sync_copy(fv, out_hbm.at[wid])


def _topk_sc(y):
    B, N = y.shape
    mesh = plsc.VectorSubcoreMesh(core_axis_name="c", subcore_axis_name="s")
    run = pl.kernel(
        functools.partial(_topk_sc_body, n=N, nb=B),
        out_type=jax.ShapeDtypeStruct((B, 64), jnp.float32),
        mesh=mesh,
        scratch_types=[
            pltpu.VMEM((N,), jnp.float32),
            pltpu.VMEM((64,), jnp.float32),
        ],
    )
    return run(y)


def _merge_top2(v1, v2, neg):
    iota = lax.broadcasted_iota(jnp.int32, v1.shape, 1)
    m1 = jnp.max(v1, axis=1, keepdims=True)
    idx = jnp.min(jnp.where(v1 == m1, iota, v1.shape[1]), axis=1,
                  keepdims=True)
    v1m = jnp.where(iota == idx, neg, v1)
    m2 = jnp.maximum(jnp.max(v1m, axis=1, keepdims=True),
                     jnp.max(v2, axis=1, keepdims=True))
    return m1, m2


def _mlp_kernel(f_ref, w1t_ref, b1_ref, w2t_ref, b2_ref, w3_ref, b3_ref,
                out_ref):
    f = f_ref[...]
    max1, max2 = _merge_top2(f[:, 0:16], f[:, 16:32], _NEG)
    nmin1, nmin2 = _merge_top2(-f[:, 32:48], -f[:, 48:64], _NEG)
    min1 = -nmin1
    min2 = -nmin2
    h1 = jax.nn.sigmoid(min1 * w1t_ref[0:1, :] + min2 * w1t_ref[1:2, :]
                        + max1 * w1t_ref[2:3, :] + max2 * w1t_ref[3:4, :]
                        + b1_ref[...])
    h2 = jax.nn.sigmoid(
        jnp.dot(h1, w2t_ref[...], preferred_element_type=jnp.float32)
        + b2_ref[...])
    o = jax.nn.sigmoid(jnp.sum(h2 * w3_ref[...], axis=1, keepdims=True)
                       + b3_ref[0, 0])
    out_ref[...] = o


def _mlp(feats, w1, b1, w2, b2, w3, b3):
    B = feats.shape[0]
    out = pl.pallas_call(
        _mlp_kernel,
        out_shape=jax.ShapeDtypeStruct((B, 1), jnp.float32),
    )(feats, w1.T, b1.reshape(1, 200), w2.T, b2.reshape(1, 100), w3,
      b3.reshape(1, 1))
    return out.reshape(-1)


def kernel(x, conv_w, conv_b, w1, b1, w2, b2, w3, b3):
    y = _conv(x, conv_w, conv_b)
    feats = _topk_sc(y)
    return _mlp(feats, w1, b1, w2, b2, w3, b3)


# speedup vs baseline: 1.0088x; 1.0088x over previous
import functools

import jax
import jax.numpy as jnp
from jax import lax
from jax.experimental import pallas as pl
from jax.experimental.pallas import tpu as pltpu
from jax.experimental.pallas import tpu_sc as plsc

NSTREAM = 8
_NEG = float("-inf")
_POS = float("inf")


def _conv_kernel(*refs, n):
    x_refs = refs[:NSTREAM]
    w_refs = refs[NSTREAM:2 * NSTREAM]
    cb_ref, out_ref = refs[2 * NSTREAM:]

    t = jnp.zeros((3, n), jnp.float32)
    for s in range(NSTREAM):
        t = t + jnp.dot(w_refs[s][...], x_refs[s][0, 0],
                        preferred_element_type=jnp.float32)

    t0 = t[0:1, :]
    t1 = t[1:2, :]
    t2 = t[2:3, :]
    zero = jnp.zeros((1, 1), jnp.float32)
    y = t1 + cb_ref[0, 0]
    y = y + jnp.concatenate([zero, t0[:, : n - 1]], axis=1)
    y = y + jnp.concatenate([t2[:, 1:], zero], axis=1)
    out_ref[...] = y.reshape(1, 1, n)


def _conv(x, conv_w, conv_b):
    B, C, N = x.shape
    cs = C // NSTREAM
    xs = x.reshape(B, NSTREAM, cs, N)
    wt = conv_w[0].T

    def x_spec(s):
        return pl.BlockSpec((1, 1, cs, N), lambda b, s=s: (b, s, 0, 0))

    def w_spec(s):
        return pl.BlockSpec((3, cs), lambda b, s=s: (0, s))

    in_specs = ([x_spec(s) for s in range(NSTREAM)]
                + [w_spec(s) for s in range(NSTREAM)]
                + [pl.BlockSpec((1, 1), lambda b: (0, 0))])
    operands = [xs] * NSTREAM + [wt] * NSTREAM + [conv_b.reshape(1, 1)]
    y = pl.pallas_call(
        functools.partial(_conv_kernel, n=N),
        grid=(B,),
        in_specs=in_specs,
        out_specs=pl.BlockSpec((1, 1, N), lambda b: (b, 0, 0)),
        out_shape=jax.ShapeDtypeStruct((B, 1, N), jnp.float32),
        compiler_params=pltpu.CompilerParams(
            dimension_semantics=("arbitrary",)),
    )(*operands)
    return y.reshape(B, N)


_NPART = 4
_UNROLL = 4


def _topk_sc_body(y_hbm, out_hbm, yv, fv, n=2048, nb=8):
    wid = lax.axis_index("s") * 2 + lax.axis_index("c")
    batch = lax.rem(wid, nb)
    part = lax.div(wid, nb)
    npn = n // _NPART
    pltpu.sync_copy(y_hbm.at[batch, pl.ds(part * npn, npn)], yv)

    def step(i, carry):
        mx1, mx2, mn1, mn2 = carry
        for u in range(_UNROLL):
            v = yv[pl.ds((i * _UNROLL + u) * 16, 16)]
            mx2 = jnp.maximum(mx2, jnp.minimum(mx1, v))
            mx1 = jnp.maximum(mx1, v)
            mn2 = jnp.minimum(mn2, jnp.maximum(mn1, v))
            mn1 = jnp.minimum(mn1, v)
        return mx1, mx2, mn1, mn2

    neg = jnp.full((16,), _NEG, jnp.float32)
    pos = jnp.full((16,), _POS, jnp.float32)
    mx1, mx2, mn1, mn2 = lax.fori_loop(0, npn // (16 * _UNROLL), step,
                                       (neg, neg, pos, pos))
    fv[pl.ds(0, 16)] = mx1
    fv[pl.ds(16, 16)] = mx2
    fv[pl.ds(32, 16)] = mn1
    fv[pl.ds(48, 16)] = mn2
    pltpu.sync_copy(fv, out_hbm.at[batch, pl.ds(part * 64, 64)])


def _topk_sc(y):
    B, N = y.shape
    mesh = plsc.VectorSubcoreMesh(core_axis_name="c", subcore_axis_name="s")
    run = pl.kernel(
        functools.partial(_topk_sc_body, n=N, nb=B),
        out_type=jax.ShapeDtypeStruct((B, 64 * _NPART), jnp.float32),
        mesh=mesh,
        scratch_types=[
            pltpu.VMEM((N // _NPART,), jnp.float32),
            pltpu.VMEM((64,), jnp.float32),
        ],
    )
    return run(y)


def _merge_top2(v1, v2, neg):
    iota = lax.broadcasted_iota(jnp.int32, v1.shape, 1)
    m1 = jnp.max(v1, axis=1, keepdims=True)
    idx = jnp.min(jnp.where(v1 == m1, iota, v1.shape[1]), axis=1,
                  keepdims=True)
    v1m = jnp.where(iota == idx, neg, v1)
    m2 = jnp.maximum(jnp.max(v1m, axis=1, keepdims=True),
                     jnp.max(v2, axis=1, keepdims=True))
    return m1, m2


def _mlp_kernel(f_ref, w1t_ref, b1_ref, w2t_ref, b2_ref, w3_ref, b3_ref,
                out_ref):
    f = f_ref[...]
    mx1 = jnp.concatenate([f[:, p * 64:p * 64 + 16] for p in range(_NPART)],
                          axis=1)
    mx2 = jnp.concatenate([f[:, p * 64 + 16:p * 64 + 32]
                           for p in range(_NPART)], axis=1)
    mn1 = jnp.concatenate([f[:, p * 64 + 32:p * 64 + 48]
                           for p in range(_NPART)], axis=1)
    mn2 = jnp.concatenate([f[:, p * 64 + 48:p * 64 + 64]
                           for p in range(_NPART)], axis=1)
    max1, max2 = _merge_top2(mx1, mx2, _NEG)
    nmin1, nmin2 = _merge_top2(-mn1, -mn2, _NEG)
    min1 = -nmin1
    min2 = -nmin2
    h1 = jax.nn.sigmoid(min1 * w1t_ref[0:1, :] + min2 * w1t_ref[1:2, :]
                        + max1 * w1t_ref[2:3, :] + max2 * w1t_ref[3:4, :]
                        + b1_ref[...])
    h2 = jax.nn.sigmoid(
        jnp.dot(h1, w2t_ref[...], preferred_element_type=jnp.float32)
        + b2_ref[...])
    o = jax.nn.sigmoid(jnp.sum(h2 * w3_ref[...], axis=1, keepdims=True)
                       + b3_ref[0, 0])
    out_ref[...] = o


def _mlp(feats, w1, b1, w2, b2, w3, b3):
    B = feats.shape[0]
    out = pl.pallas_call(
        _mlp_kernel,
        out_shape=jax.ShapeDtypeStruct((B, 1), jnp.float32),
    )(feats, w1.T, b1.reshape(1, 200), w2.T, b2.reshape(1, 100), w3,
      b3.reshape(1, 1))
    return out.reshape(-1)


def kernel(x, conv_w, conv_b, w1, b1, w2, b2, w3, b3):
    y = _conv(x, conv_w, conv_b)
    feats = _topk_sc(y)
    return _mlp(feats, w1, b1, w2, b2, w3, b3)


# speedup vs baseline: 1.4649x; 1.4521x over previous
import functools

import jax
import jax.numpy as jnp
from jax.experimental import pallas as pl
from jax.experimental.pallas import tpu as pltpu

NSTREAM = 8


def _chowder_kernel(*refs, n):
    x_refs = refs[:NSTREAM]
    w_refs = refs[NSTREAM:2 * NSTREAM]
    (cb_ref, w1t_ref, b1_ref, w2t_ref, b2_ref, w3_ref, b3_ref, out_ref) = \
        refs[2 * NSTREAM:]

    t = jnp.zeros((3, n), jnp.float32)
    for s in range(NSTREAM):
        t = t + jnp.dot(w_refs[s][...], x_refs[s][0, 0],
                        preferred_element_type=jnp.float32)

    t0 = t[0:1, :]
    t1 = t[1:2, :]
    t2 = t[2:3, :]
    zero = jnp.zeros((1, 1), jnp.float32)
    y = t1 + cb_ref[0, 0]
    y = y + jnp.concatenate([zero, t0[:, : n - 1]], axis=1)
    y = y + jnp.concatenate([t2[:, 1:], zero], axis=1)

    max1 = jnp.max(y)
    mmax = y == max1
    nmax = jnp.sum(mmax.astype(jnp.float32))
    max_rest = jnp.max(jnp.where(mmax, -jnp.inf, y))
    max2 = jnp.where(nmax > 1.5, max1, max_rest)
    min1 = jnp.min(y)
    mmin = y == min1
    nmin = jnp.sum(mmin.astype(jnp.float32))
    min_rest = jnp.min(jnp.where(mmin, jnp.inf, y))
    min2 = jnp.where(nmin > 1.5, min1, min_rest)

    h1 = jax.nn.sigmoid(min1 * w1t_ref[0:1, :] + min2 * w1t_ref[1:2, :]
                        + max1 * w1t_ref[2:3, :] + max2 * w1t_ref[3:4, :]
                        + b1_ref[...])
    h2 = jax.nn.sigmoid(
        jnp.dot(h1, w2t_ref[...], preferred_element_type=jnp.float32)
        + b2_ref[...])
    o = jax.nn.sigmoid(jnp.sum(h2 * w3_ref[...]) + b3_ref[0, 0])
    out_ref[...] = o.reshape(1, 1, 1)


def kernel(x, conv_w, conv_b, w1, b1, w2, b2, w3, b3):
    B, C, N = x.shape
    cs = C // NSTREAM
    xs = x.reshape(B, NSTREAM, cs, N)
    wt = conv_w[0].T

    def x_spec(s):
        return pl.BlockSpec((1, 1, cs, N), lambda b, s=s: (b, s, 0, 0))

    def w_spec(s):
        return pl.BlockSpec((3, cs), lambda b, s=s: (0, s))

    in_specs = ([x_spec(s) for s in range(NSTREAM)]
                + [w_spec(s) for s in range(NSTREAM)]
                + [
        pl.BlockSpec((1, 1), lambda b: (0, 0)),
        pl.BlockSpec((4, 200), lambda b: (0, 0)),
        pl.BlockSpec((1, 200), lambda b: (0, 0)),
        pl.BlockSpec((200, 100), lambda b: (0, 0)),
        pl.BlockSpec((1, 100), lambda b: (0, 0)),
        pl.BlockSpec((1, 100), lambda b: (0, 0)),
        pl.BlockSpec((1, 1), lambda b: (0, 0)),
    ])
    operands = ([xs] * NSTREAM + [wt] * NSTREAM
                + [conv_b.reshape(1, 1), w1.T, b1.reshape(1, 200), w2.T,
                   b2.reshape(1, 100), w3, b3.reshape(1, 1)])
    out = pl.pallas_call(
        functools.partial(_chowder_kernel, n=N),
        grid=(B,),
        in_specs=in_specs,
        out_specs=pl.BlockSpec((1, 1, 1), lambda b: (b, 0, 0)),
        out_shape=jax.ShapeDtypeStruct((B, 1, 1), jnp.float32),
        compiler_params=pltpu.CompilerParams(
            dimension_semantics=("arbitrary",)),
    )(*operands)
    return out.reshape(-1)
